# Initial kernel scaffold; baseline (speedup 1.0000x reference)
import jax
import jax.numpy as jnp
from jax.experimental import pallas as pl


def kernel(h, r, pos_t, neg_t, entity_embed, relation_embed, trans_M):
    raise NotImplementedError("write your pallas kernel here")


# speedup vs baseline: 1.6132x; 1.6132x over previous
import functools

import jax
import jax.numpy as jnp
from jax import lax
from jax.experimental import pallas as pl
from jax.experimental.pallas import tpu as pltpu
from jax.experimental.pallas import tpu_sc as plsc

N_REL = 64
ED = 64
RD = 64
B = 4096
L2_LAMBDA = 1e-05

NW = 32
BPW = B // NW

BB = 512
NB = B // BB


def _sc_gather(entity_embed, relation_embed, h, r, pos_t, neg_t):
    mesh = plsc.VectorSubcoreMesh(core_axis_name="c", subcore_axis_name="s")

    @functools.partial(
        pl.kernel,
        out_type=[jax.ShapeDtypeStruct((B, ED), jnp.float32) for _ in range(4)],
        mesh=mesh,
        scratch_types=[
            pltpu.VMEM((BPW,), jnp.int32),
            pltpu.VMEM((BPW, ED), jnp.float32),
            pltpu.SemaphoreType.DMA,
        ],
        compiler_params=pltpu.CompilerParams(use_tc_tiling_on_sc=False),
    )
    def gather_k(ent_hbm, rel_hbm, h_hbm, r_hbm, p_hbm, n_hbm,
                 out_h, out_r, out_p, out_n, idx_v, rows_v, sem):
        wid = lax.axis_index("s") * 2 + lax.axis_index("c")
        base = wid * BPW
        for idx_hbm, tab, out in ((h_hbm, ent_hbm, out_h),
                                  (r_hbm, rel_hbm, out_r),
                                  (p_hbm, ent_hbm, out_p),
                                  (n_hbm, ent_hbm, out_n)):
            pltpu.sync_copy(idx_hbm.at[pl.ds(base, BPW)], idx_v)
            pltpu.async_copy(tab.at[idx_v], rows_v, sem).wait()
            pltpu.sync_copy(rows_v, out.at[pl.ds(base, BPW)])

    return gather_k(entity_embed, relation_embed, h, r, pos_t, neg_t)


def _tc_body(h_ref, p_ref, n_ref, re_ref, r_ref, m_ref, out_ref, acc_ref):
    @pl.when(pl.program_id(0) == 0)
    def _init():
        acc_ref[0] = 0.0
        acc_ref[1] = 0.0

    w3 = m_ref[...].reshape(N_REL * ED, RD)
    rcol = r_ref[...]
    lane_k = lax.broadcasted_iota(jnp.int32, (BB, N_REL * ED), 1) // ED
    maskf = (lane_k == rcol).astype(jnp.float32)

    def proj(x):
        xt = jnp.tile(x, (1, N_REL))
        return lax.dot_general(xt * maskf, w3, (((1,), (0,)), ((), ())),
                               preferred_element_type=jnp.float32)

    rh = proj(h_ref[...])
    rp = proj(p_ref[...])
    rn = proj(n_ref[...])
    re = re_ref[...]

    anchor = rh + re
    pos_s = jnp.sum(jnp.square(anchor - rp), axis=1, keepdims=True)
    neg_s = jnp.sum(jnp.square(anchor - rn), axis=1, keepdims=True)
    d = neg_s - pos_s
    trip = jnp.maximum(-d, 0.0) + jnp.log(1.0 + jnp.exp(-jnp.abs(d)))
    l2 = 0.5 * (jnp.sum(jnp.square(rh)) + jnp.sum(jnp.square(re))
                + jnp.sum(jnp.square(rp)) + jnp.sum(jnp.square(rn)))

    acc_ref[0] += jnp.sum(trip)
    acc_ref[1] += l2

    @pl.when(pl.program_id(0) == NB - 1)
    def _fin():
        loss = acc_ref[0] / B + L2_LAMBDA * (acc_ref[1] / B)
        out_ref[...] = jnp.full((1, 1), loss, dtype=jnp.float32)


def _tc_loss(head, pos, neg, remb, r2, trans_M):
    return pl.pallas_call(
        _tc_body,
        grid=(NB,),
        in_specs=[
            pl.BlockSpec((BB, ED), lambda i: (i, 0)),
            pl.BlockSpec((BB, ED), lambda i: (i, 0)),
            pl.BlockSpec((BB, ED), lambda i: (i, 0)),
            pl.BlockSpec((BB, RD), lambda i: (i, 0)),
            pl.BlockSpec((BB, 1), lambda i: (i, 0)),
            pl.BlockSpec((N_REL, ED, RD), lambda i: (0, 0, 0)),
        ],
        out_specs=pl.BlockSpec((1, 1), lambda i: (0, 0)),
        out_shape=jax.ShapeDtypeStruct((1, 1), jnp.float32),
        scratch_shapes=[pltpu.SMEM((2,), jnp.float32)],
        compiler_params=pltpu.CompilerParams(
            dimension_semantics=("arbitrary",)),
    )(head, pos, neg, remb, r2, trans_M)


def kernel(h, r, pos_t, neg_t, entity_embed, relation_embed, trans_M):
    h = h.astype(jnp.int32)
    r = r.astype(jnp.int32)
    pos_t = pos_t.astype(jnp.int32)
    neg_t = neg_t.astype(jnp.int32)
    head, remb, pos, neg = _sc_gather(entity_embed, relation_embed,
                                      h, r, pos_t, neg_t)
    out = _tc_loss(head, pos, neg, remb, r.reshape(B, 1), trans_M)
    return out[0, 0]
